# baseline (speedup 1.0000x reference)
import functools

import jax
import jax.numpy as jnp
from jax import lax
from jax.experimental import pallas as pl
from jax.experimental.pallas import tpu as pltpu
from jax.experimental.pallas import tpu_sc as plsc

BOARD_LEN = 10
N_BOARD_CELL = 100
N_OCC = 92
N_PIECE_ID = 82
N_CH = 96
N_OUT_CH = N_CH + N_PIECE_ID
N_WORKERS = 32
L = 16
BC = 512
GPC = BC // L


def _cell_of(j):
    off = ((j >= 42).astype(jnp.int32) + (j >= 44).astype(jnp.int32)
           + (j >= 48).astype(jnp.int32) + (j >= 50).astype(jnp.int32))
    return j + 2 * off


def _sc_kernel(x_hbm, p_hbm, out_hbm,
               p_t, o_t0, o_t1, sv0, sv1, sem_ch, so0, so1):
    batch = x_hbm.shape[3]
    n_chunks = batch // BC
    wid = lax.axis_index("s") * 2 + lax.axis_index("c")

    three = wid < 28
    nj = jnp.where(three, 3, 2)
    j0 = jnp.where(three, 3 * wid, 84 + 2 * (wid - 28))

    os_, svs, sos = (o_t0, o_t1), (sv0, sv1), (so0, so1)
    lane = jnp.arange(L, dtype=jnp.int32)
    zeros = jnp.zeros((L,), jnp.float32)
    ones = jnp.ones((L,), jnp.float32)

    for par in (0, 1):
        def zrow(r, _, o_t=os_[par]):
            rows = jnp.full((L,), r, jnp.int32)
            for g in range(GPC):
                plsc.store_scatter(o_t, [rows, lane + L * g], zeros)
            return 0
        lax.fori_loop(0, N_PIECE_ID, zrow, 0)

    def per_cell(jj, _):
        j = j0 + jj
        cell = _cell_of(j)
        r, c = cell // BOARD_LEN, cell % BOARD_LEN

        pltpu.async_copy(x_hbm.at[r, c, :, :], out_hbm.at[j, 0:N_CH, :],
                         sem_ch)
        pltpu.sync_copy(p_hbm.at[r, c, :], p_t)

        def chunk_pair(k2, _):
            for par in (0, 1):
                k = 2 * k2 + par
                o_t, sv = os_[par], svs[par]
                g_abs = jj * n_chunks + k

                @pl.when(g_abs >= 2)
                def _():
                    pltpu.make_async_copy(
                        o_t, out_hbm.at[j, N_CH:N_OUT_CH, pl.ds(k * BC, BC)],
                        sos[par]).wait()
                    for g in range(GPC):
                        pv_old = sv[pl.ds(g * L, L)]
                        plsc.store_scatter(o_t, [pv_old, lane + L * g], zeros)

                for g in range(GPC):
                    pv = p_t[pl.ds(k * BC + g * L, L)]
                    sv[pl.ds(g * L, L)] = pv
                    plsc.store_scatter(o_t, [pv, lane + L * g], ones)

                pltpu.async_copy(
                    o_t, out_hbm.at[j, N_CH:N_OUT_CH, pl.ds(k * BC, BC)],
                    sos[par])
            return 0

        lax.fori_loop(0, n_chunks // 2, chunk_pair, 0)
        return 0

    lax.fori_loop(0, nj, per_cell, 0)

    for par in (0, 1):
        pltpu.make_async_copy(
            os_[par], out_hbm.at[0, N_CH:N_OUT_CH, pl.ds(0, BC)],
            sos[par]).wait()

    def drain_ch(jj, _):
        pltpu.make_async_copy(x_hbm.at[0, 0, :, :], out_hbm.at[0, 0:N_CH, :],
                              sem_ch).wait()
        return 0
    lax.fori_loop(0, nj, drain_ch, 0)


def kernel(infostate_tensor, piece_ids, piece_id_onehot):
    del piece_id_onehot
    B = infostate_tensor.shape[0]
    xt = jnp.transpose(infostate_tensor, (2, 3, 1, 0))
    pt = jnp.transpose(piece_ids.astype(jnp.int32), (1, 2, 0))

    run = functools.partial(
        pl.kernel,
        out_type=jax.ShapeDtypeStruct((N_OCC, N_OUT_CH, B), jnp.float32),
        mesh=plsc.VectorSubcoreMesh(core_axis_name="c", subcore_axis_name="s"),
        compiler_params=pltpu.CompilerParams(
            use_tc_tiling_on_sc=True, needs_layout_passes=False
        ),
        scratch_types=[
            pltpu.VMEM((4096,), jnp.int32),
            pltpu.VMEM((N_PIECE_ID, BC), jnp.float32),
            pltpu.VMEM((N_PIECE_ID, BC), jnp.float32),
            pltpu.VMEM((BC,), jnp.int32),
            pltpu.VMEM((BC,), jnp.int32),
            pltpu.SemaphoreType.DMA,
            pltpu.SemaphoreType.DMA,
            pltpu.SemaphoreType.DMA,
        ],
    )(_sc_kernel)
    out = run(xt, pt)
    return jnp.transpose(out, (2, 0, 1))


# speedup vs baseline: 25.3913x; 25.3913x over previous
import functools

import jax
import jax.numpy as jnp
from jax import lax
from jax.experimental import pallas as pl
from jax.experimental.pallas import tpu as pltpu
from jax.experimental.pallas import tpu_sc as plsc

BOARD_LEN = 10
N_OCC = 92
N_PIECE_ID = 82
N_CH = 96
N_OUT_CH = N_CH + N_PIECE_ID
L = 16
BC = 128
GPC = BC // L
CPJ_SHIFT = 5


def _cell_of(j):
    off = ((j >= 42).astype(jnp.int32) + (j >= 44).astype(jnp.int32)
           + (j >= 48).astype(jnp.int32) + (j >= 50).astype(jnp.int32))
    return j + 2 * off


def _rc_of(j):
    cell = _cell_of(j)
    return cell // BOARD_LEN, cell % BOARD_LEN


def _sc_kernel(x_hbm, p_hbm, out_hbm, p_t,
               cb0, cb1, cb2, cb3, o_t0, o_t1, sv0, sv1,
               si0, si1, si2, si3, sc0, sc1, sc2, sc3, so0, so1):
    batch = x_hbm.shape[3]
    cpj = batch // BC
    wid = lax.axis_index("s") * 2 + lax.axis_index("c")

    three = wid < 28
    nj = jnp.where(three, 3, 2)
    j0 = jnp.where(three, 3 * wid, 84 + 2 * (wid - 28))
    total = nj * cpj

    cbs = (cb0, cb1, cb2, cb3)
    sis, scs = (si0, si1, si2, si3), (sc0, sc1, sc2, sc3)
    os_, svs, sos = (o_t0, o_t1), (sv0, sv1), (so0, so1)
    lane = jnp.arange(L, dtype=jnp.int32)
    zeros = jnp.zeros((L,), jnp.float32)
    ones = jnp.ones((L,), jnp.float32)

    def chunk_coords(g):
        jj = lax.shift_right_logical(g, CPJ_SHIFT)
        k = lax.bitwise_and(g, cpj - 1)
        return j0 + jj, k

    def start_chan_in(g, buf, sem):
        j, k = chunk_coords(g)
        r, c = _rc_of(j)
        pltpu.async_copy(x_hbm.at[r, c, :, pl.ds(k * BC, BC)], buf, sem)

    for par in (0, 1):
        def zrow(rr, _, o_t=os_[par]):
            rows = jnp.full((L,), rr, jnp.int32)
            for g in range(GPC):
                plsc.store_scatter(o_t, [rows, lane + L * g], zeros)
            return 0
        lax.fori_loop(0, N_PIECE_ID, zrow, 0)

    for i in (0, 1):
        start_chan_in(i, cbs[i], sis[i])

    def quad(q, _):
        for i in range(4):
            g = 4 * q + i
            p2 = i & 1
            o_t, sv = os_[p2], svs[p2]
            j, k = chunk_coords(g)
            r, c = _rc_of(j)

            @pl.when(g < total)
            def _():
                @pl.when(k == 0)
                def _():
                    pltpu.sync_copy(p_hbm.at[r, c, :], p_t)

                nxt = (i + 2) & 3

                @pl.when(g + 2 < total)
                def _():
                    @pl.when(g >= 2)
                    def _():
                        pltpu.make_async_copy(
                            cbs[nxt], out_hbm.at[j, 0:N_CH, pl.ds(0, BC)],
                            scs[nxt]).wait()
                    start_chan_in(g + 2, cbs[nxt], sis[nxt])

                @pl.when(g >= 2)
                def _():
                    pltpu.make_async_copy(
                        o_t, out_hbm.at[j, N_CH:N_OUT_CH, pl.ds(0, BC)],
                        sos[p2]).wait()
                    for gg in range(GPC):
                        pv_old = sv[pl.ds(gg * L, L)]
                        plsc.store_scatter(o_t, [pv_old, lane + L * gg],
                                           zeros)

                for gg in range(GPC):
                    pv = p_t[pl.ds(k * BC + gg * L, L)]
                    sv[pl.ds(gg * L, L)] = pv
                    plsc.store_scatter(o_t, [pv, lane + L * gg], ones)

                pltpu.async_copy(
                    o_t, out_hbm.at[j, N_CH:N_OUT_CH, pl.ds(k * BC, BC)],
                    sos[p2])

                pltpu.make_async_copy(
                    x_hbm.at[r, c, :, pl.ds(k * BC, BC)], cbs[i],
                    sis[i]).wait()
                pltpu.async_copy(
                    cbs[i], out_hbm.at[j, 0:N_CH, pl.ds(k * BC, BC)], scs[i])
        return 0

    lax.fori_loop(0, (total + 3) // 4, quad, 0)

    for i in range(4):
        pltpu.make_async_copy(
            cbs[i], out_hbm.at[0, 0:N_CH, pl.ds(0, BC)], scs[i]).wait()
    for par in (0, 1):
        pltpu.make_async_copy(
            os_[par], out_hbm.at[0, N_CH:N_OUT_CH, pl.ds(0, BC)],
            sos[par]).wait()


def kernel(infostate_tensor, piece_ids, piece_id_onehot):
    del piece_id_onehot
    B = infostate_tensor.shape[0]
    xt = jnp.transpose(infostate_tensor, (2, 3, 1, 0))
    pt = jnp.transpose(piece_ids.astype(jnp.int32), (1, 2, 0))

    run = functools.partial(
        pl.kernel,
        out_type=jax.ShapeDtypeStruct((N_OCC, N_OUT_CH, B), jnp.float32),
        mesh=plsc.VectorSubcoreMesh(core_axis_name="c", subcore_axis_name="s"),
        compiler_params=pltpu.CompilerParams(
            use_tc_tiling_on_sc=True, needs_layout_passes=False
        ),
        scratch_types=[
            pltpu.VMEM((B,), jnp.int32),
            pltpu.VMEM((N_CH, BC), jnp.float32),
            pltpu.VMEM((N_CH, BC), jnp.float32),
            pltpu.VMEM((N_CH, BC), jnp.float32),
            pltpu.VMEM((N_CH, BC), jnp.float32),
            pltpu.VMEM((N_PIECE_ID, BC), jnp.float32),
            pltpu.VMEM((N_PIECE_ID, BC), jnp.float32),
            pltpu.VMEM((BC,), jnp.int32),
            pltpu.VMEM((BC,), jnp.int32),
            pltpu.SemaphoreType.DMA,
            pltpu.SemaphoreType.DMA,
            pltpu.SemaphoreType.DMA,
            pltpu.SemaphoreType.DMA,
            pltpu.SemaphoreType.DMA,
            pltpu.SemaphoreType.DMA,
            pltpu.SemaphoreType.DMA,
            pltpu.SemaphoreType.DMA,
            pltpu.SemaphoreType.DMA,
            pltpu.SemaphoreType.DMA,
        ],
    )(_sc_kernel)
    out = run(xt, pt)
    return jnp.transpose(out, (2, 0, 1))
